# baseline (device time: 44366 ns/iter reference)
import jax
import jax.numpy as jnp
from jax import lax
from jax.experimental import pallas as pl
from jax.experimental.pallas import tpu as pltpu

CHUNKS = (64, 160, 160, 160, 128, 64, 16, 16)
NC = len(CHUNKS)
OFFS = tuple(sum(CHUNKS[:i]) for i in range(NC))
CHMAX = max(CHUNKS)


def kernel(A, B):
    M, K = A.shape
    _, N = B.shape
    HALF = M // 2
    assert sum(CHUNKS) == HALF

    def body(a_ref, b_ref, out_ref,
             x_send, x_recv, y_send, y_recv,
             x_send_sems, x_recv_sems, y_send_sems, y_recv_sems, copy_sems):
        my_x = lax.axis_index("x")
        my_y = lax.axis_index("y")
        x_peer = (1 - my_x, my_y)
        y_peer = (my_x, 1 - my_y)

        barrier_sem = pltpu.get_barrier_semaphore()
        for peer in (x_peer, y_peer):
            pl.semaphore_signal(
                barrier_sem, inc=1, device_id=peer,
                device_id_type=pl.DeviceIdType.MESH,
            )

        def x_rdma(c):
            sub = (c, pl.ds(0, CHUNKS[c]))
            return pltpu.make_async_remote_copy(
                src_ref=x_send.at[sub], dst_ref=x_recv.at[sub],
                send_sem=x_send_sems.at[c], recv_sem=x_recv_sems.at[c],
                device_id=x_peer, device_id_type=pl.DeviceIdType.MESH,
            )

        def y_rdma(c):
            sub = (c, pl.ds(0, CHUNKS[c]))
            return pltpu.make_async_remote_copy(
                src_ref=y_send.at[sub], dst_ref=y_recv.at[sub],
                send_sem=y_send_sems.at[c], recv_sem=y_recv_sems.at[c],
                device_id=y_peer, device_id_type=pl.DeviceIdType.MESH,
            )

        def out_copy(src, c, half_owner, sem):
            rows = pl.ds(half_owner * HALF + OFFS[c], CHUNKS[c])
            return pltpu.make_async_copy(
                src.at[c, pl.ds(0, CHUNKS[c])], out_ref.at[rows, :],
                copy_sems.at[sem],
            )

        b_bf16 = b_ref[...].astype(jnp.bfloat16)

        partials = []
        for c in range(NC):
            rows = pl.ds(my_y * HALF + OFFS[c], CHUNKS[c])
            p = jnp.dot(
                a_ref[rows, :].astype(jnp.bfloat16), b_bf16,
                preferred_element_type=jnp.float32,
            )
            partials.append(p)
            x_send[c, pl.ds(0, CHUNKS[c])] = p.astype(jnp.bfloat16)
            if c == 0:
                pl.semaphore_wait(barrier_sem, 2)
            x_rdma(c).start()

        for c in range(NC):
            x_rdma(c).wait_recv()
            red = partials[c] + x_recv[c, pl.ds(0, CHUNKS[c])].astype(jnp.float32)
            y_send[c, pl.ds(0, CHUNKS[c])] = red.astype(jnp.bfloat16)
            y_rdma(c).start()
            out_copy(y_send, c, my_y, c).start()

        for c in range(NC):
            y_rdma(c).wait_recv()
            out_copy(y_recv, c, 1 - my_y, NC + c).start()

        for c in range(NC):
            out_copy(y_send, c, my_y, c).wait()
        for c in range(NC):
            out_copy(y_recv, c, 1 - my_y, NC + c).wait()
        for c in range(NC):
            x_rdma(c).wait_send()
            y_rdma(c).wait_send()

    return pl.pallas_call(
        body,
        out_shape=jax.ShapeDtypeStruct((M, N), jnp.bfloat16),
        in_specs=[
            pl.BlockSpec(memory_space=pltpu.VMEM),
            pl.BlockSpec(memory_space=pltpu.VMEM),
        ],
        out_specs=pl.BlockSpec(memory_space=pltpu.MemorySpace.HBM),
        scratch_shapes=[
            pltpu.VMEM((NC, CHMAX, N), jnp.bfloat16),
            pltpu.VMEM((NC, CHMAX, N), jnp.bfloat16),
            pltpu.VMEM((NC, CHMAX, N), jnp.bfloat16),
            pltpu.VMEM((NC, CHMAX, N), jnp.bfloat16),
            pltpu.SemaphoreType.DMA((NC,)),
            pltpu.SemaphoreType.DMA((NC,)),
            pltpu.SemaphoreType.DMA((NC,)),
            pltpu.SemaphoreType.DMA((NC,)),
            pltpu.SemaphoreType.DMA((2 * NC,)),
        ],
        compiler_params=pltpu.CompilerParams(collective_id=0),
    )(A, B)
